# initial kernel scaffold (unmeasured)
import jax
import jax.numpy as jnp
from jax import lax
from jax.experimental import pallas as pl
from jax.experimental.pallas import tpu as pltpu

N_DEV = 32


def kernel(x, w_mat, scale_x, scale_w):
    m, _ = x.shape
    _, n = w_mat.shape
    chunk = m // N_DEV

    def body(x_ref, w_ref, sx_ref, sw_ref, out_ref, comm_ref,
             send_sems, recv_sems, credit_sem):
        my = lax.axis_index("i")
        left = lax.rem(my - 1 + N_DEV, N_DEV)
        right = lax.rem(my + 1, N_DEV)

        barrier_sem = pltpu.get_barrier_semaphore()
        for nbr in (left, right):
            pl.semaphore_signal(barrier_sem, inc=1, device_id=(nbr,),
                                device_id_type=pl.DeviceIdType.MESH)
        pl.semaphore_wait(barrier_sem, 2)

        scale = sx_ref[0] * sw_ref[0]
        acc = lax.dot_general(
            x_ref[...], w_ref[...],
            dimension_numbers=(((1,), (0,)), ((), ())),
            preferred_element_type=jnp.float32,
        )
        out_ref[...] = acc * scale

        n_steps = 2 * (N_DEV - 1)
        for u in range(n_steps):
            slot = u % 2
            if u < N_DEV - 1:
                s = u
                send_idx = lax.rem(my - s + N_DEV, N_DEV)
                recv_idx = lax.rem(my - s - 1 + N_DEV, N_DEV)
            else:
                t = u - (N_DEV - 1)
                send_idx = lax.rem(my + 1 - t + 2 * N_DEV, N_DEV)
                recv_idx = lax.rem(my - t + 2 * N_DEV, N_DEV)

            if u >= 2:
                pl.semaphore_wait(credit_sem, 1)

            rdma = pltpu.make_async_remote_copy(
                src_ref=out_ref.at[pl.ds(send_idx * chunk, chunk), :],
                dst_ref=comm_ref.at[slot],
                send_sem=send_sems.at[slot],
                recv_sem=recv_sems.at[slot],
                device_id=(right,),
                device_id_type=pl.DeviceIdType.MESH,
            )
            rdma.start()
            rdma.wait()

            if u < N_DEV - 1:
                out_ref[pl.ds(recv_idx * chunk, chunk), :] = (
                    out_ref[pl.ds(recv_idx * chunk, chunk), :]
                    + comm_ref[slot]
                )
            else:
                out_ref[pl.ds(recv_idx * chunk, chunk), :] = comm_ref[slot]

            if u < n_steps - 2:
                pl.semaphore_signal(credit_sem, inc=1, device_id=(left,),
                                    device_id_type=pl.DeviceIdType.MESH)

    return pl.pallas_call(
        body,
        out_shape=jax.ShapeDtypeStruct((m, n), jnp.float32),
        in_specs=[
            pl.BlockSpec(memory_space=pltpu.VMEM),
            pl.BlockSpec(memory_space=pltpu.VMEM),
            pl.BlockSpec(memory_space=pltpu.SMEM),
            pl.BlockSpec(memory_space=pltpu.SMEM),
        ],
        out_specs=pl.BlockSpec(memory_space=pltpu.VMEM),
        scratch_shapes=[
            pltpu.VMEM((2, chunk, n), jnp.float32),
            pltpu.SemaphoreType.DMA((2,)),
            pltpu.SemaphoreType.DMA((2,)),
            pltpu.SemaphoreType.REGULAR,
        ],
        compiler_params=pltpu.CompilerParams(collective_id=0),
    )(x, w_mat, scale_x, scale_w)


# baseline (device time: 862154 ns/iter reference)
import jax
import jax.numpy as jnp
from jax import lax
from jax.experimental import pallas as pl
from jax.experimental.pallas import tpu as pltpu

N_DEV = 32


def kernel(x, w_mat, scale_x, scale_w):
    m, _ = x.shape
    _, n = w_mat.shape
    chunk = m // N_DEV

    def body(x_ref, w_ref, sx_ref, sw_ref, out_ref, comm_ref,
             send_sems, recv_sems, credit_sem):
        my = lax.axis_index("i")
        left = lax.rem(my - 1 + N_DEV, N_DEV)
        right = lax.rem(my + 1, N_DEV)

        barrier_sem = pltpu.get_barrier_semaphore()
        for nbr in (left, right):
            pl.semaphore_signal(barrier_sem, inc=1, device_id=(nbr,),
                                device_id_type=pl.DeviceIdType.MESH)
        pl.semaphore_wait(barrier_sem, 2)

        scale = sx_ref[0] * sw_ref[0]
        acc = lax.dot_general(
            x_ref[...], w_ref[...],
            dimension_numbers=(((1,), (0,)), ((), ())),
            preferred_element_type=jnp.float32,
        )
        out_ref[...] = acc * scale

        n_steps = 2 * (N_DEV - 1)
        for u in range(n_steps):
            slot = u % 2
            if u < N_DEV - 1:
                s = u
                send_idx = lax.rem(my - s + N_DEV, N_DEV)
                recv_idx = lax.rem(my - s - 1 + N_DEV, N_DEV)
            else:
                t = u - (N_DEV - 1)
                send_idx = lax.rem(my + 1 - t + 2 * N_DEV, N_DEV)
                recv_idx = lax.rem(my - t + 2 * N_DEV, N_DEV)

            if u >= 2:
                pl.semaphore_wait(credit_sem, 1)

            rdma = pltpu.make_async_remote_copy(
                src_ref=out_ref.at[pl.ds(send_idx * chunk, chunk), :],
                dst_ref=comm_ref.at[slot],
                send_sem=send_sems.at[slot],
                recv_sem=recv_sems.at[slot],
                device_id=(right,),
                device_id_type=pl.DeviceIdType.MESH,
            )
            rdma.start()
            rdma.wait()

            if u < N_DEV - 1:
                out_ref[pl.ds(recv_idx * chunk, chunk), :] = (
                    out_ref[pl.ds(recv_idx * chunk, chunk), :]
                    + comm_ref[slot]
                )
            else:
                out_ref[pl.ds(recv_idx * chunk, chunk), :] = comm_ref[slot]

            if u < n_steps - 2:
                pl.semaphore_signal(credit_sem, inc=1, device_id=(left,),
                                    device_id_type=pl.DeviceIdType.MESH)

    return pl.pallas_call(
        body,
        out_shape=jax.ShapeDtypeStruct((m, n), jnp.float32),
        in_specs=[
            pl.BlockSpec(memory_space=pltpu.VMEM),
            pl.BlockSpec(memory_space=pltpu.VMEM),
            pl.BlockSpec(memory_space=pltpu.SMEM),
            pl.BlockSpec(memory_space=pltpu.SMEM),
        ],
        out_specs=pl.BlockSpec(memory_space=pltpu.VMEM),
        scratch_shapes=[
            pltpu.VMEM((2, chunk, n), jnp.float32),
            pltpu.SemaphoreType.DMA((2,)),
            pltpu.SemaphoreType.DMA((2,)),
            pltpu.SemaphoreType.REGULAR,
        ],
        compiler_params=pltpu.CompilerParams(
            collective_id=0,
            vmem_limit_bytes=100 * 1024 * 1024,
        ),
    )(x, w_mat, scale_x, scale_w)


# device time: 846089 ns/iter; 1.0190x vs baseline; 1.0190x over previous
import jax
import jax.numpy as jnp
from jax import lax
from jax.experimental import pallas as pl
from jax.experimental.pallas import tpu as pltpu

N_DEV = 32


def kernel(x, w_mat, scale_x, scale_w):
    m, _ = x.shape
    _, n = w_mat.shape
    chunk = m // N_DEV
    half = chunk // 2

    def body(x_ref, w_ref, sx_ref, sw_ref, out_ref,
             comm_r, comm_l, send_sems_r, recv_sems_r,
             send_sems_l, recv_sems_l, credit_r, credit_l):
        my = lax.axis_index("i")
        left = lax.rem(my - 1 + N_DEV, N_DEV)
        right = lax.rem(my + 1, N_DEV)

        barrier_sem = pltpu.get_barrier_semaphore()
        for nbr in (left, right):
            pl.semaphore_signal(barrier_sem, inc=1, device_id=(nbr,),
                                device_id_type=pl.DeviceIdType.MESH)
        pl.semaphore_wait(barrier_sem, 2)

        scale = sx_ref[0] * sw_ref[0]
        acc = lax.dot_general(
            x_ref[...], w_ref[...],
            dimension_numbers=(((1,), (0,)), ((), ())),
            preferred_element_type=jnp.float32,
        )
        out_ref[...] = acc * scale

        def top_rows(idx):
            return pl.ds(idx * chunk, half)

        def bot_rows(idx):
            return pl.ds(idx * chunk + half, half)

        n_steps = 2 * (N_DEV - 1)
        for u in range(n_steps):
            slot = u % 2
            if u < N_DEV - 1:
                s = u
                send_r_idx = lax.rem(my - s + N_DEV, N_DEV)
                recv_r_idx = lax.rem(my - s - 1 + N_DEV, N_DEV)
                send_l_idx = lax.rem(my + s, N_DEV)
                recv_l_idx = lax.rem(my + s + 1, N_DEV)
            else:
                t = u - (N_DEV - 1)
                send_r_idx = lax.rem(my + 1 - t + 2 * N_DEV, N_DEV)
                recv_r_idx = lax.rem(my - t + 2 * N_DEV, N_DEV)
                send_l_idx = lax.rem(my - 1 + t + N_DEV, N_DEV)
                recv_l_idx = lax.rem(my + t, N_DEV)

            if u >= 2:
                pl.semaphore_wait(credit_r, 1)
                pl.semaphore_wait(credit_l, 1)

            rdma_r = pltpu.make_async_remote_copy(
                src_ref=out_ref.at[top_rows(send_r_idx), :],
                dst_ref=comm_r.at[slot],
                send_sem=send_sems_r.at[slot],
                recv_sem=recv_sems_r.at[slot],
                device_id=(right,),
                device_id_type=pl.DeviceIdType.MESH,
            )
            rdma_l = pltpu.make_async_remote_copy(
                src_ref=out_ref.at[bot_rows(send_l_idx), :],
                dst_ref=comm_l.at[slot],
                send_sem=send_sems_l.at[slot],
                recv_sem=recv_sems_l.at[slot],
                device_id=(left,),
                device_id_type=pl.DeviceIdType.MESH,
            )
            rdma_r.start()
            rdma_l.start()

            rdma_r.wait()
            if u < N_DEV - 1:
                out_ref[top_rows(recv_r_idx), :] = (
                    out_ref[top_rows(recv_r_idx), :] + comm_r[slot]
                )
            else:
                out_ref[top_rows(recv_r_idx), :] = comm_r[slot]

            rdma_l.wait()
            if u < N_DEV - 1:
                out_ref[bot_rows(recv_l_idx), :] = (
                    out_ref[bot_rows(recv_l_idx), :] + comm_l[slot]
                )
            else:
                out_ref[bot_rows(recv_l_idx), :] = comm_l[slot]

            if u < n_steps - 2:
                pl.semaphore_signal(credit_r, inc=1, device_id=(left,),
                                    device_id_type=pl.DeviceIdType.MESH)
                pl.semaphore_signal(credit_l, inc=1, device_id=(right,),
                                    device_id_type=pl.DeviceIdType.MESH)

    return pl.pallas_call(
        body,
        out_shape=jax.ShapeDtypeStruct((m, n), jnp.float32),
        in_specs=[
            pl.BlockSpec(memory_space=pltpu.VMEM),
            pl.BlockSpec(memory_space=pltpu.VMEM),
            pl.BlockSpec(memory_space=pltpu.SMEM),
            pl.BlockSpec(memory_space=pltpu.SMEM),
        ],
        out_specs=pl.BlockSpec(memory_space=pltpu.VMEM),
        scratch_shapes=[
            pltpu.VMEM((2, half, n), jnp.float32),
            pltpu.VMEM((2, half, n), jnp.float32),
            pltpu.SemaphoreType.DMA((2,)),
            pltpu.SemaphoreType.DMA((2,)),
            pltpu.SemaphoreType.DMA((2,)),
            pltpu.SemaphoreType.DMA((2,)),
            pltpu.SemaphoreType.REGULAR,
            pltpu.SemaphoreType.REGULAR,
        ],
        compiler_params=pltpu.CompilerParams(
            collective_id=0,
            vmem_limit_bytes=100 * 1024 * 1024,
        ),
    )(x, w_mat, scale_x, scale_w)


# device time: 749177 ns/iter; 1.1508x vs baseline; 1.1294x over previous
import jax
import jax.numpy as jnp
from jax import lax
from jax.experimental import pallas as pl
from jax.experimental.pallas import tpu as pltpu

N_DEV = 32
N_FLOW = 2


def kernel(x, w_mat, scale_x, scale_w):
    m, _ = x.shape
    _, n = w_mat.shape
    chunk = m // N_DEV
    half = chunk // N_FLOW

    def body(x_ref, w_ref, sx_ref, sw_ref, out_ref,
             comm0, comm1, send_sems0, recv_sems0, send_sems1, recv_sems1,
             credit0, credit1):
        comms = (comm0, comm1)
        send_sems = (send_sems0, send_sems1)
        recv_sems = (recv_sems0, recv_sems1)
        credits = (credit0, credit1)

        my = lax.axis_index("i")
        left = lax.rem(my - 1 + N_DEV, N_DEV)
        right = lax.rem(my + 1, N_DEV)

        barrier_sem = pltpu.get_barrier_semaphore()
        for nbr in (left, right):
            pl.semaphore_signal(barrier_sem, inc=1, device_id=(nbr,),
                                device_id_type=pl.DeviceIdType.MESH)
        pl.semaphore_wait(barrier_sem, 2)

        scale = sx_ref[0] * sw_ref[0]
        acc = lax.dot_general(
            x_ref[...] * scale, w_ref[...],
            dimension_numbers=(((1,), (0,)), ((), ())),
            preferred_element_type=jnp.float32,
        )
        out_ref[...] = acc

        n_steps = 2 * (N_DEV - 1)

        def indices(u):
            if u < N_DEV - 1:
                s = u
                return (lax.rem(my - s + N_DEV, N_DEV),
                        lax.rem(my - s - 1 + N_DEV, N_DEV))
            t = u - (N_DEV - 1)
            return (lax.rem(my + 1 - t + 2 * N_DEV, N_DEV),
                    lax.rem(my - t + 2 * N_DEV, N_DEV))

        def rows(idx, h):
            return pl.ds(idx * chunk + h * half, half)

        def launch(u, h):
            send_idx, _ = indices(u)
            if u >= 2:
                pl.semaphore_wait(credits[h], 1)
            rdma = pltpu.make_async_remote_copy(
                src_ref=out_ref.at[rows(send_idx, h), :],
                dst_ref=comms[h].at[u % 2],
                send_sem=send_sems[h].at[u % 2],
                recv_sem=recv_sems[h].at[u % 2],
                device_id=(right,),
                device_id_type=pl.DeviceIdType.MESH,
            )
            rdma.start()
            return rdma

        def finish(u, h, rdma):
            _, recv_idx = indices(u)
            rdma.wait()
            if u < N_DEV - 1:
                out_ref[rows(recv_idx, h), :] = (
                    out_ref[rows(recv_idx, h), :] + comms[h][u % 2]
                )
            else:
                out_ref[rows(recv_idx, h), :] = comms[h][u % 2]
            if u < n_steps - 2:
                pl.semaphore_signal(credits[h], inc=1, device_id=(left,),
                                    device_id_type=pl.DeviceIdType.MESH)

        pending = [launch(0, h) for h in range(N_FLOW)]
        for u in range(1, n_steps):
            for h in range(N_FLOW):
                finish(u - 1, h, pending[h])
                pending[h] = launch(u, h)
        for h in range(N_FLOW):
            finish(n_steps - 1, h, pending[h])

    return pl.pallas_call(
        body,
        out_shape=jax.ShapeDtypeStruct((m, n), jnp.float32),
        in_specs=[
            pl.BlockSpec(memory_space=pltpu.VMEM),
            pl.BlockSpec(memory_space=pltpu.VMEM),
            pl.BlockSpec(memory_space=pltpu.SMEM),
            pl.BlockSpec(memory_space=pltpu.SMEM),
        ],
        out_specs=pl.BlockSpec(memory_space=pltpu.VMEM),
        scratch_shapes=[
            pltpu.VMEM((2, half, n), jnp.float32),
            pltpu.VMEM((2, half, n), jnp.float32),
            pltpu.SemaphoreType.DMA((2,)),
            pltpu.SemaphoreType.DMA((2,)),
            pltpu.SemaphoreType.DMA((2,)),
            pltpu.SemaphoreType.DMA((2,)),
            pltpu.SemaphoreType.REGULAR,
            pltpu.SemaphoreType.REGULAR,
        ],
        compiler_params=pltpu.CompilerParams(
            collective_id=0,
            vmem_limit_bytes=100 * 1024 * 1024,
        ),
    )(x, w_mat, scale_x, scale_w)


# device time: 401961 ns/iter; 2.1449x vs baseline; 1.8638x over previous
import jax
import jax.numpy as jnp
from jax import lax
from jax.experimental import pallas as pl
from jax.experimental.pallas import tpu as pltpu

N_DEV = 32
N_FLOW = 4


def kernel(x, w_mat, scale_x, scale_w):
    m, _ = x.shape
    _, n = w_mat.shape
    chunk = m // N_DEV
    sub = chunk // N_FLOW

    def body(x_ref, w_ref, sx_ref, sw_ref, out_ref,
             comm, stage, send_sems, recv_sems, credits):
        my = lax.axis_index("i")
        left = lax.rem(my - 1 + N_DEV, N_DEV)
        right = lax.rem(my + 1, N_DEV)

        barrier_sem = pltpu.get_barrier_semaphore()
        for nbr in (left, right):
            pl.semaphore_signal(barrier_sem, inc=1, device_id=(nbr,),
                                device_id_type=pl.DeviceIdType.MESH)
        pl.semaphore_wait(barrier_sem, 2)

        scale = sx_ref[0] * sw_ref[0]
        acc = lax.dot_general(
            x_ref[...] * scale, w_ref[...],
            dimension_numbers=(((1,), (0,)), ((), ())),
            preferred_element_type=jnp.float32,
        )
        out_ref[...] = acc

        n_steps = 2 * (N_DEV - 1)

        def indices(u):
            if u < N_DEV - 1:
                s = u
                return (lax.rem(my - s + N_DEV, N_DEV),
                        lax.rem(my - s - 1 + N_DEV, N_DEV))
            t = u - (N_DEV - 1)
            return (lax.rem(my + 1 - t + 2 * N_DEV, N_DEV),
                    lax.rem(my - t + 2 * N_DEV, N_DEV))

        def rows(idx, h):
            return pl.ds(idx * chunk + h * sub, sub)

        def launch(u, h):
            send_idx, _ = indices(u)
            if u >= 2:
                pl.semaphore_wait(credits.at[h], 1)
            stage[h, u % 2] = out_ref[rows(send_idx, h), :].astype(
                jnp.bfloat16)
            rdma = pltpu.make_async_remote_copy(
                src_ref=stage.at[h, u % 2],
                dst_ref=comm.at[h, u % 2],
                send_sem=send_sems.at[h, u % 2],
                recv_sem=recv_sems.at[h, u % 2],
                device_id=(right,),
                device_id_type=pl.DeviceIdType.MESH,
            )
            rdma.start()
            return rdma

        def finish(u, h, rdma):
            _, recv_idx = indices(u)
            rdma.wait()
            inbound = comm[h, u % 2].astype(jnp.float32)
            if u < N_DEV - 1:
                out_ref[rows(recv_idx, h), :] = (
                    out_ref[rows(recv_idx, h), :] + inbound
                )
            else:
                out_ref[rows(recv_idx, h), :] = inbound
            if u < n_steps - 2:
                pl.semaphore_signal(credits.at[h], inc=1, device_id=(left,),
                                    device_id_type=pl.DeviceIdType.MESH)

        pending = [launch(0, h) for h in range(N_FLOW)]
        for u in range(1, n_steps):
            for h in range(N_FLOW):
                finish(u - 1, h, pending[h])
                pending[h] = launch(u, h)
        for h in range(N_FLOW):
            finish(n_steps - 1, h, pending[h])

    return pl.pallas_call(
        body,
        out_shape=jax.ShapeDtypeStruct((m, n), jnp.float32),
        in_specs=[
            pl.BlockSpec(memory_space=pltpu.VMEM),
            pl.BlockSpec(memory_space=pltpu.VMEM),
            pl.BlockSpec(memory_space=pltpu.SMEM),
            pl.BlockSpec(memory_space=pltpu.SMEM),
        ],
        out_specs=pl.BlockSpec(memory_space=pltpu.VMEM),
        scratch_shapes=[
            pltpu.VMEM((N_FLOW, 2, sub, n), jnp.bfloat16),
            pltpu.VMEM((N_FLOW, 2, sub, n), jnp.bfloat16),
            pltpu.SemaphoreType.DMA((N_FLOW, 2)),
            pltpu.SemaphoreType.DMA((N_FLOW, 2)),
            pltpu.SemaphoreType.REGULAR((N_FLOW,)),
        ],
        compiler_params=pltpu.CompilerParams(
            collective_id=0,
            vmem_limit_bytes=100 * 1024 * 1024,
        ),
    )(x, w_mat, scale_x, scale_w)
